# baseline (speedup 1.0000x reference)
import functools

import jax
import jax.numpy as jnp
from jax import lax
from jax.experimental import pallas as pl
from jax.experimental.pallas import tpu as pltpu
from jax.experimental.pallas import tpu_sc as plsc

N = 50000
E = 800000
F_IN = 128
NODE_D = 112
GRAPH_D = 16
H = 4
C = 8
EDGE_D = 4
OUT = 8
G = 64
HC = H * C

NPAD = 50048
ZSL = NPAD // 16
CH = 128
CPT = 392
EPAD = 16 * CPT * CH
BN = 1000
GRID_N = N // BN


def _ea_stats_body(ea_ref, st_ref):
    i = pl.program_id(0)
    blk = ea_ref[...]
    m = jnp.max(jnp.abs(blk), axis=0, keepdims=True)
    s = jnp.sum(blk, axis=0, keepdims=True)

    @pl.when(i == 0)
    def _():
        st_ref[...] = jnp.concatenate([m, s], axis=1)

    @pl.when(i > 0)
    def _():
        st_ref[...] = jnp.concatenate(
            [jnp.maximum(st_ref[:, :EDGE_D], m), st_ref[:, EDGE_D:] + s], axis=1)


def _ea_stats(ea):
    eb = 8000
    return pl.pallas_call(
        _ea_stats_body,
        grid=(E // eb,),
        in_specs=[pl.BlockSpec((eb, EDGE_D), lambda i: (i, 0))],
        out_specs=pl.BlockSpec((1, 2 * EDGE_D), lambda i: (0, 0)),
        out_shape=jax.ShapeDtypeStruct((1, 2 * EDGE_D), jnp.float32),
    )(ea)


def _prep_body(x_ref, w_ref, p_ref, s8_ref, bm_ref):
    r = jnp.dot(x_ref[...], w_ref[...], preferred_element_type=jnp.float32)
    p_ref[...] = r
    s8_ref[...] = r[:, HC:HC + 16]
    bm_ref[...] = jnp.max(r[:, HC:HC + 8], axis=0, keepdims=True)[None]


def _prep(x_cur, wfull):
    din = x_cur.shape[1]
    return pl.pallas_call(
        _prep_body,
        grid=(GRID_N,),
        in_specs=[pl.BlockSpec((BN, din), lambda i: (i, 0)),
                  pl.BlockSpec((din, 48), lambda i: (0, 0))],
        out_specs=[pl.BlockSpec((BN, 48), lambda i: (i, 0)),
                   pl.BlockSpec((BN, 16), lambda i: (i, 0)),
                   pl.BlockSpec((1, 1, 8), lambda i: (i, 0, 0))],
        out_shape=[jax.ShapeDtypeStruct((N, 48), jnp.float32),
                   jax.ShapeDtypeStruct((N, 16), jnp.float32),
                   jax.ShapeDtypeStruct((GRID_N, 1, 8), jnp.float32)],
    )(x_cur, wfull)


def _k_body(s8_ref, aux_ref, dst_ref, exl_ref):
    s8 = s8_ref[...]
    als = s8[:, :H]
    ald = s8[:, H:2 * H]
    bub = aux_ref[0, :H]
    alf = aux_ref[0, H:2 * H]
    beta_self = als + alf[None, :]
    kz = bub[None, :] + ald
    kv = jnp.where(kz > 0, kz, 0.2 * kz)
    zl = beta_self + ald
    al_loop = jnp.where(zl > 0, zl, 0.2 * zl)
    exl = jnp.exp(al_loop - kv)
    zpad = jnp.zeros_like(als)
    dst_ref[...] = jnp.concatenate([ald, kv, zpad, zpad], axis=1)
    exl_ref[...] = jnp.concatenate([exl, zpad], axis=1)


def _k_kernel(s8, aux):
    kb = 1024
    grid = pl.cdiv(NPAD, kb)
    return pl.pallas_call(
        _k_body,
        grid=(grid,),
        in_specs=[pl.BlockSpec((kb, 16), lambda i: (i, 0)),
                  pl.BlockSpec((1, 16), lambda i: (0, 0))],
        out_specs=[pl.BlockSpec((kb, 16), lambda i: (i, 0)),
                   pl.BlockSpec((kb, 8), lambda i: (i, 0))],
        out_shape=[jax.ShapeDtypeStruct((NPAD, 16), jnp.float32),
                   jax.ShapeDtypeStruct((NPAD, 8), jnp.float32)],
    )(s8, aux)


def _combine12_body(a_ref, e_ref, p_ref, b_ref, x_ref):
    hmat = p_ref[:, :HC]
    exl = e_ref[...]
    outs = []
    for h in range(H):
        core, loc = divmod(h, 2)
        num = a_ref[core][:, loc * C:(loc + 1) * C]
        den = a_ref[core][:, 2 * C + loc:2 * C + loc + 1]
        el = exl[:, h:h + 1]
        hh = hmat[:, h * C:(h + 1) * C]
        outs.append((num + el * hh) / (den + el))
    o = jnp.concatenate(outs, axis=1) + b_ref[0][None, :]
    x_ref[...] = jnp.maximum(o, 0.0)


def _combine12(accp, exl, p, b):
    return pl.pallas_call(
        _combine12_body,
        grid=(GRID_N,),
        in_specs=[pl.BlockSpec((2, BN, 32), lambda i: (0, i, 0)),
                  pl.BlockSpec((BN, 8), lambda i: (i, 0)),
                  pl.BlockSpec((BN, 48), lambda i: (i, 0)),
                  pl.BlockSpec((1, HC), lambda i: (0, 0))],
        out_specs=pl.BlockSpec((BN, HC), lambda i: (i, 0)),
        out_shape=jax.ShapeDtypeStruct((N, HC), jnp.float32),
    )(accp, exl, p, b)


def _combine3_body(a_ref, e_ref, p_ref, b_ref, f_ref):
    hmat = p_ref[:, :HC]
    exl = e_ref[...]
    acc = None
    for h in range(H):
        core, loc = divmod(h, 2)
        num = a_ref[core][:, loc * C:(loc + 1) * C]
        den = a_ref[core][:, 2 * C + loc:2 * C + loc + 1]
        el = exl[:, h:h + 1]
        hh = hmat[:, h * C:(h + 1) * C]
        o = (num + el * hh) / (den + el)
        acc = o if acc is None else acc + o
    f_ref[...] = acc * (1.0 / H) + b_ref[0][None, :]


def _combine3(accp, exl, p, b3):
    return pl.pallas_call(
        _combine3_body,
        grid=(GRID_N,),
        in_specs=[pl.BlockSpec((2, BN, 32), lambda i: (0, i, 0)),
                  pl.BlockSpec((BN, 8), lambda i: (i, 0)),
                  pl.BlockSpec((BN, 48), lambda i: (i, 0)),
                  pl.BlockSpec((1, C), lambda i: (0, 0))],
        out_specs=pl.BlockSpec((BN, C), lambda i: (i, 0)),
        out_shape=jax.ShapeDtypeStruct((N, C), jnp.float32),
    )(accp, exl, p, b3)


def _pool_body(f_ref, gf_ref, ids_ref, wm1_ref, bm1_ref, wm2_ref, bm2_ref,
               wm3_ref, bm3_ref, o_ref, ps_ref):
    i = pl.program_id(0)
    ids = ids_ref[0]
    giota = lax.broadcasted_iota(jnp.int32, (G, BN), 0)
    oh = (giota == ids).astype(jnp.float32)
    feat = jnp.concatenate(
        [f_ref[...], gf_ref[...], jnp.ones((BN, 1), jnp.float32)], axis=1)
    part = jnp.dot(oh, feat, preferred_element_type=jnp.float32)

    @pl.when(i == 0)
    def _():
        ps_ref[...] = part

    @pl.when(i > 0)
    def _():
        ps_ref[...] = ps_ref[...] + part

    @pl.when(i == GRID_N - 1)
    def _():
        ps = ps_ref[...]
        cnt = jnp.maximum(ps[:, 24:25], 1.0)
        z = ps[:, :24] / cnt
        z = jnp.maximum(jnp.dot(z, wm1_ref[...], preferred_element_type=jnp.float32)
                        + bm1_ref[0][None, :], 0.0)
        z = jnp.maximum(jnp.dot(z, wm2_ref[...], preferred_element_type=jnp.float32)
                        + bm2_ref[0][None, :], 0.0)
        o_ref[...] = (jnp.dot(z, wm3_ref[...], preferred_element_type=jnp.float32)
                      + bm3_ref[0][None, :])


def _pool_mlp(f, gf, ids3, Wm1, bm1, Wm2, bm2, Wm3, bm3):
    return pl.pallas_call(
        _pool_body,
        grid=(GRID_N,),
        in_specs=[pl.BlockSpec((BN, C), lambda i: (i, 0)),
                  pl.BlockSpec((BN, GRAPH_D), lambda i: (i, 0)),
                  pl.BlockSpec((1, 1, BN), lambda i: (i, 0, 0)),
                  pl.BlockSpec((C + GRAPH_D, C), lambda i: (0, 0)),
                  pl.BlockSpec((1, C), lambda i: (0, 0)),
                  pl.BlockSpec((C, C // 2), lambda i: (0, 0)),
                  pl.BlockSpec((1, C // 2), lambda i: (0, 0)),
                  pl.BlockSpec((C // 2, OUT), lambda i: (0, 0)),
                  pl.BlockSpec((1, OUT), lambda i: (0, 0))],
        out_specs=pl.BlockSpec((G, OUT), lambda i: (0, 0)),
        out_shape=jax.ShapeDtypeStruct((G, OUT), jnp.float32),
        scratch_shapes=[pltpu.VMEM((G, 25), jnp.float32)],
    )(f, gf, ids3, Wm1, bm1, Wm2, bm2, Wm3, bm3)


def _splat_dyn(ref, base, off):
    return plsc.load_gather(ref, [jnp.full((16,), base, jnp.int32) + off])


NB = 2


def _pass_b_body(src_hbm, dst_hbm, ea_hbm, p_hbm, dstt_hbm, weab_hbm, z_hbm,
                 out_hbm, srcv, dstv, eav, pbuf, dbuf, msg, weav, acc,
                 semp, semd, sems):
    cid = lax.axis_index("c")
    sid = lax.axis_index("s")
    pltpu.sync_copy(z_hbm.at[pl.ds(sid * ZSL, ZSL)], acc.at[pl.ds(sid * ZSL, ZSL)])
    pltpu.sync_copy(weab_hbm, weav)

    def zrow(i, _):
        for b in range(NB):
            msg[b][i, pl.ds(0, 16)] = jnp.zeros((16,), jnp.float32)
            msg[b][i, pl.ds(16, 16)] = jnp.zeros((16,), jnp.float32)
        return 0

    lax.fori_loop(0, CH, zrow, 0)
    plsc.subcore_barrier()

    iota = lax.iota(jnp.int32, 16)
    hbase = cid * 2
    wsp = [[_splat_dyn(weav, j * H, hbase + hh) for j in range(EDGE_D)]
           for hh in range(2)]

    def superchunk(i, _):
        cps = []
        for b in range(NB):
            base = (sid * CPT + i * NB + b) * CH
            pltpu.sync_copy(src_hbm.at[pl.ds(base, CH)], srcv[b])
            pltpu.sync_copy(dst_hbm.at[pl.ds(base, CH)], dstv[b])
            pltpu.sync_copy(ea_hbm.at[pl.ds(base, CH)], eav[b])
            cps.append((pltpu.async_copy(p_hbm.at[srcv[b]], pbuf[b], semp[b]),
                        pltpu.async_copy(dstt_hbm.at[dstv[b]], dbuf[b], semd[b])))
        scs = []
        for b in range(NB):
            cps[b][0].wait()
            cps[b][1].wait()

            def group(g, _, b=b):
                rows = iota + g * 16
                eaj = [plsc.load_gather(
                    eav[b], [rows, jnp.full((16,), j, jnp.int32)])
                    for j in range(EDGE_D)]
                for hh in range(2):
                    hcol = hbase + hh
                    als = plsc.load_gather(
                        pbuf[b], [rows, jnp.full((16,), HC, jnp.int32) + hcol])
                    ald = plsc.load_gather(
                        dbuf[b], [rows, jnp.full((16,), 0, jnp.int32) + hcol])
                    kv = plsc.load_gather(
                        dbuf[b], [rows, jnp.full((16,), H, jnp.int32) + hcol])
                    ale = eaj[0] * wsp[hh][0]
                    for j in range(1, EDGE_D):
                        ale = ale + eaj[j] * wsp[hh][j]
                    z = als + ald + ale
                    alpha = jnp.where(z > 0, z, 0.2 * z)
                    ex = jnp.exp(alpha - kv)
                    plsc.store_scatter(
                        msg[b], [rows, jnp.full((16,), 2 * C + hh, jnp.int32)],
                        ex)
                    for c in range(C):
                        hv = plsc.load_gather(
                            pbuf[b],
                            [rows, jnp.full((16,), c, jnp.int32) + hcol * C])
                        plsc.store_scatter(
                            msg[b],
                            [rows, jnp.full((16,), hh * C + c, jnp.int32)],
                            ex * hv)
                return 0

            lax.fori_loop(0, CH // 16, group, 0)
            pltpu.sync_copy(msg[b], acc.at[dstv[b]], add=True)
        return 0

    lax.fori_loop(0, CPT // NB, superchunk, 0)
    plsc.subcore_barrier()
    pltpu.sync_copy(acc.at[pl.ds(sid * ZSL, ZSL)],
                    out_hbm.at[cid, pl.ds(sid * ZSL, ZSL)])


@functools.lru_cache(maxsize=1)
def _sc_kernels():
    mesh = plsc.VectorSubcoreMesh(core_axis_name="c", subcore_axis_name="s")
    cp = pltpu.CompilerParams(needs_layout_passes=False,
                              use_tc_tiling_on_sc=False)
    pass_b = pl.kernel(
        _pass_b_body,
        out_type=jax.ShapeDtypeStruct((2, NPAD, 32), jnp.float32),
        mesh=mesh,
        compiler_params=cp,
        scratch_types=[
            [pltpu.VMEM((CH,), jnp.int32)] * NB,
            [pltpu.VMEM((CH,), jnp.int32)] * NB,
            [pltpu.VMEM((CH, EDGE_D), jnp.float32)] * NB,
            [pltpu.VMEM((CH, 48), jnp.float32)] * NB,
            [pltpu.VMEM((CH, 16), jnp.float32)] * NB,
            [pltpu.VMEM((CH, 32), jnp.float32)] * NB,
            pltpu.VMEM((32,), jnp.float32),
            pltpu.VMEM_SHARED((NPAD, 32), jnp.float32),
            [pltpu.SemaphoreType.DMA] * NB,
            [pltpu.SemaphoreType.DMA] * NB,
            [pltpu.SemaphoreType.DMA] * NB,
        ],
    )
    return pass_b


def _fold(a):
    idx = jnp.arange(HC)
    return jnp.zeros((HC, H), jnp.float32).at[idx, idx // C].set(a.reshape(HC))


def kernel(x, edge_index, edge_attr, batch,
           W1, as1, ad1, We1, ae1, b1,
           W2, as2, ad2, We2, ae2, b2,
           W3, as3, ad3, We3, ae3, b3,
           Wm1, bm1, Wm2, bm2, Wm3, bm3):
    src = edge_index[0].astype(jnp.int32)
    dst = edge_index[1].astype(jnp.int32)
    src_p = jnp.concatenate([src, jnp.zeros((EPAD - E,), jnp.int32)])
    dst_p = jnp.concatenate([dst, jnp.full((EPAD - E,), N, jnp.int32)])
    ea_p = jnp.concatenate(
        [edge_attr, jnp.zeros((EPAD - E, EDGE_D), jnp.float32)])

    st = _ea_stats(edge_attr)
    m_abs = st[0, :EDGE_D]
    fill = st[0, EDGE_D:] / E

    z32 = jnp.zeros((NPAD, 32), jnp.float32)

    pass_b = _sc_kernels()
    layers = [(W1, as1, ad1, We1, ae1, b1),
              (W2, as2, ad2, We2, ae2, b2),
              (W3, as3, ad3, We3, ae3, b3)]
    x_cur = x[:, :NODE_D]
    f3 = None
    for li, (W, a_s, a_d, We, a_e, b) in enumerate(layers):
        wfull = jnp.concatenate(
            [W, W @ _fold(a_s), W @ _fold(a_d),
             jnp.zeros((W.shape[0], 8), jnp.float32)], axis=1)
        we_a = jnp.einsum('jhc,hc->jh', We.reshape(EDGE_D, H, C), a_e)
        p_tab, s8, bm = _prep(x_cur, wfull)
        alsmax = jnp.max(bm[:, 0, :H], axis=0)
        ub = jnp.sum(m_abs[:, None] * jnp.abs(we_a), axis=0)
        bub = alsmax + ub
        al_fill = fill @ we_a
        weab = jnp.concatenate(
            [we_a.reshape(-1), bub, jnp.zeros((12,), jnp.float32)])
        aux = jnp.concatenate(
            [bub, al_fill, jnp.zeros((8,), jnp.float32)]).reshape(1, 16)
        dstt, exl = _k_kernel(s8, aux)
        accp = pass_b(src_p, dst_p, ea_p, p_tab, dstt, weab, z32)
        if li < 2:
            x_cur = _combine12(accp, exl, p_tab, b.reshape(1, HC))
        else:
            f3 = _combine3(accp, exl, p_tab, b.reshape(1, C))

    gf = x[:, NODE_D:]
    ids3 = batch.astype(jnp.int32).reshape(GRID_N, 1, BN)
    return _pool_mlp(f3, gf, ids3,
                     Wm1, bm1.reshape(1, C),
                     Wm2, bm2.reshape(1, C // 2),
                     Wm3, bm3.reshape(1, OUT))


# speedup vs baseline: 1.4821x; 1.4821x over previous
import functools

import jax
import jax.numpy as jnp
from jax import lax
from jax.experimental import pallas as pl
from jax.experimental.pallas import tpu as pltpu
from jax.experimental.pallas import tpu_sc as plsc

N = 50000
E = 800000
F_IN = 128
NODE_D = 112
GRAPH_D = 16
H = 4
C = 8
EDGE_D = 4
OUT = 8
G = 64
HC = H * C

NPAD = 50048
ZSL = NPAD // 16
CH = 512
CPT = 98
EPAD = 16 * CPT * CH
BN = 1000
GRID_N = N // BN


def _ea_stats_body(ea_ref, st_ref):
    i = pl.program_id(0)
    blk = ea_ref[...]
    m = jnp.max(jnp.abs(blk), axis=0, keepdims=True)
    s = jnp.sum(blk, axis=0, keepdims=True)

    @pl.when(i == 0)
    def _():
        st_ref[...] = jnp.concatenate([m, s], axis=1)

    @pl.when(i > 0)
    def _():
        st_ref[...] = jnp.concatenate(
            [jnp.maximum(st_ref[:, :EDGE_D], m), st_ref[:, EDGE_D:] + s], axis=1)


def _ea_stats(ea):
    eb = 8000
    return pl.pallas_call(
        _ea_stats_body,
        grid=(E // eb,),
        in_specs=[pl.BlockSpec((eb, EDGE_D), lambda i: (i, 0))],
        out_specs=pl.BlockSpec((1, 2 * EDGE_D), lambda i: (0, 0)),
        out_shape=jax.ShapeDtypeStruct((1, 2 * EDGE_D), jnp.float32),
    )(ea)


def _prep_body(x_ref, w_ref, p_ref, s8_ref, bm_ref):
    r = jnp.dot(x_ref[...], w_ref[...], preferred_element_type=jnp.float32)
    p_ref[...] = r
    s8_ref[...] = r[:, HC:HC + 8]
    bm_ref[...] = jnp.max(r[:, HC:HC + 8], axis=0, keepdims=True)[None]


def _prep(x_cur, wfull):
    din = x_cur.shape[1]
    return pl.pallas_call(
        _prep_body,
        grid=(GRID_N,),
        in_specs=[pl.BlockSpec((BN, din), lambda i: (i, 0)),
                  pl.BlockSpec((din, 40), lambda i: (0, 0))],
        out_specs=[pl.BlockSpec((BN, 40), lambda i: (i, 0)),
                   pl.BlockSpec((BN, 8), lambda i: (i, 0)),
                   pl.BlockSpec((1, 1, 8), lambda i: (i, 0, 0))],
        out_shape=[jax.ShapeDtypeStruct((N, 40), jnp.float32),
                   jax.ShapeDtypeStruct((N, 8), jnp.float32),
                   jax.ShapeDtypeStruct((GRID_N, 1, 8), jnp.float32)],
    )(x_cur, wfull)


def _k_body(s8_ref, aux_ref, dst_ref, exl_ref):
    s8 = s8_ref[...]
    als = s8[:, :H]
    ald = s8[:, H:2 * H]
    bub = aux_ref[0, :H]
    alf = aux_ref[0, H:2 * H]
    beta_self = als + alf[None, :]
    kz = bub[None, :] + ald
    kv = jnp.where(kz > 0, kz, 0.2 * kz)
    zl = beta_self + ald
    al_loop = jnp.where(zl > 0, zl, 0.2 * zl)
    exl = jnp.exp(al_loop - kv)
    zpad = jnp.zeros_like(als)
    dst_ref[...] = jnp.concatenate([ald, kv], axis=1)
    exl_ref[...] = jnp.concatenate([exl, zpad], axis=1)


def _k_kernel(s8, aux):
    kb = 1024
    grid = pl.cdiv(NPAD, kb)
    return pl.pallas_call(
        _k_body,
        grid=(grid,),
        in_specs=[pl.BlockSpec((kb, 8), lambda i: (i, 0)),
                  pl.BlockSpec((1, 16), lambda i: (0, 0))],
        out_specs=[pl.BlockSpec((kb, 8), lambda i: (i, 0)),
                   pl.BlockSpec((kb, 8), lambda i: (i, 0))],
        out_shape=[jax.ShapeDtypeStruct((NPAD, 8), jnp.float32),
                   jax.ShapeDtypeStruct((NPAD, 8), jnp.float32)],
    )(s8, aux)


def _combine12_body(a_ref, e_ref, p_ref, b_ref, x_ref):
    hmat = p_ref[:, :HC]
    exl = e_ref[...]
    outs = []
    for h in range(H):
        core, loc = divmod(h, 2)
        num = a_ref[core][:, loc * C:(loc + 1) * C]
        den = a_ref[core][:, 2 * C + loc:2 * C + loc + 1]
        el = exl[:, h:h + 1]
        hh = hmat[:, h * C:(h + 1) * C]
        outs.append((num + el * hh) / (den + el))
    o = jnp.concatenate(outs, axis=1) + b_ref[0][None, :]
    x_ref[...] = jnp.maximum(o, 0.0)


def _combine12(accp, exl, p, b):
    return pl.pallas_call(
        _combine12_body,
        grid=(GRID_N,),
        in_specs=[pl.BlockSpec((2, BN, 24), lambda i: (0, i, 0)),
                  pl.BlockSpec((BN, 8), lambda i: (i, 0)),
                  pl.BlockSpec((BN, 40), lambda i: (i, 0)),
                  pl.BlockSpec((1, HC), lambda i: (0, 0))],
        out_specs=pl.BlockSpec((BN, HC), lambda i: (i, 0)),
        out_shape=jax.ShapeDtypeStruct((N, HC), jnp.float32),
    )(accp, exl, p, b)


def _combine3_body(a_ref, e_ref, p_ref, b_ref, f_ref):
    hmat = p_ref[:, :HC]
    exl = e_ref[...]
    acc = None
    for h in range(H):
        core, loc = divmod(h, 2)
        num = a_ref[core][:, loc * C:(loc + 1) * C]
        den = a_ref[core][:, 2 * C + loc:2 * C + loc + 1]
        el = exl[:, h:h + 1]
        hh = hmat[:, h * C:(h + 1) * C]
        o = (num + el * hh) / (den + el)
        acc = o if acc is None else acc + o
    f_ref[...] = acc * (1.0 / H) + b_ref[0][None, :]


def _combine3(accp, exl, p, b3):
    return pl.pallas_call(
        _combine3_body,
        grid=(GRID_N,),
        in_specs=[pl.BlockSpec((2, BN, 24), lambda i: (0, i, 0)),
                  pl.BlockSpec((BN, 8), lambda i: (i, 0)),
                  pl.BlockSpec((BN, 40), lambda i: (i, 0)),
                  pl.BlockSpec((1, C), lambda i: (0, 0))],
        out_specs=pl.BlockSpec((BN, C), lambda i: (i, 0)),
        out_shape=jax.ShapeDtypeStruct((N, C), jnp.float32),
    )(accp, exl, p, b3)


def _pool_body(f_ref, gf_ref, ids_ref, wm1_ref, bm1_ref, wm2_ref, bm2_ref,
               wm3_ref, bm3_ref, o_ref, ps_ref):
    i = pl.program_id(0)
    ids = ids_ref[0]
    giota = lax.broadcasted_iota(jnp.int32, (G, BN), 0)
    oh = (giota == ids).astype(jnp.float32)
    feat = jnp.concatenate(
        [f_ref[...], gf_ref[...], jnp.ones((BN, 1), jnp.float32)], axis=1)
    part = jnp.dot(oh, feat, preferred_element_type=jnp.float32)

    @pl.when(i == 0)
    def _():
        ps_ref[...] = part

    @pl.when(i > 0)
    def _():
        ps_ref[...] = ps_ref[...] + part

    @pl.when(i == GRID_N - 1)
    def _():
        ps = ps_ref[...]
        cnt = jnp.maximum(ps[:, 24:25], 1.0)
        z = ps[:, :24] / cnt
        z = jnp.maximum(jnp.dot(z, wm1_ref[...], preferred_element_type=jnp.float32)
                        + bm1_ref[0][None, :], 0.0)
        z = jnp.maximum(jnp.dot(z, wm2_ref[...], preferred_element_type=jnp.float32)
                        + bm2_ref[0][None, :], 0.0)
        o_ref[...] = (jnp.dot(z, wm3_ref[...], preferred_element_type=jnp.float32)
                      + bm3_ref[0][None, :])


def _pool_mlp(f, gf, ids3, Wm1, bm1, Wm2, bm2, Wm3, bm3):
    return pl.pallas_call(
        _pool_body,
        grid=(GRID_N,),
        in_specs=[pl.BlockSpec((BN, C), lambda i: (i, 0)),
                  pl.BlockSpec((BN, GRAPH_D), lambda i: (i, 0)),
                  pl.BlockSpec((1, 1, BN), lambda i: (i, 0, 0)),
                  pl.BlockSpec((C + GRAPH_D, C), lambda i: (0, 0)),
                  pl.BlockSpec((1, C), lambda i: (0, 0)),
                  pl.BlockSpec((C, C // 2), lambda i: (0, 0)),
                  pl.BlockSpec((1, C // 2), lambda i: (0, 0)),
                  pl.BlockSpec((C // 2, OUT), lambda i: (0, 0)),
                  pl.BlockSpec((1, OUT), lambda i: (0, 0))],
        out_specs=pl.BlockSpec((G, OUT), lambda i: (0, 0)),
        out_shape=jax.ShapeDtypeStruct((G, OUT), jnp.float32),
        scratch_shapes=[pltpu.VMEM((G, 25), jnp.float32)],
    )(f, gf, ids3, Wm1, bm1, Wm2, bm2, Wm3, bm3)


def _splat_dyn(ref, base, off):
    return plsc.load_gather(ref, [jnp.full((16,), base, jnp.int32) + off])


NB = 1


def _pass_b_body(src_hbm, dst_hbm, ea_hbm, p_hbm, dstt_hbm, weab_hbm, z_hbm,
                 out_hbm, srcv, dstv, eav, pbuf, dbuf, msg, weav, acc,
                 semp, semd, sems):
    cid = lax.axis_index("c")
    sid = lax.axis_index("s")
    pltpu.sync_copy(z_hbm.at[pl.ds(sid * ZSL, ZSL)], acc.at[pl.ds(sid * ZSL, ZSL)])
    pltpu.sync_copy(weab_hbm, weav)

    def zrow(i, _):
        for b in range(NB):
            msg[b][i, pl.ds(0, 16)] = jnp.zeros((16,), jnp.float32)
            msg[b][i, pl.ds(8, 16)] = jnp.zeros((16,), jnp.float32)
        return 0

    lax.fori_loop(0, CH, zrow, 0)
    plsc.subcore_barrier()

    iota = lax.iota(jnp.int32, 16)
    hbase = cid * 2
    wsp = [[_splat_dyn(weav, j * H, hbase + hh) for j in range(EDGE_D)]
           for hh in range(2)]

    def superchunk(i, _):
        cps = []
        for b in range(NB):
            base = (sid * CPT + i * NB + b) * CH
            pltpu.sync_copy(src_hbm.at[pl.ds(base, CH)], srcv[b])
            pltpu.sync_copy(dst_hbm.at[pl.ds(base, CH)], dstv[b])
            pltpu.sync_copy(ea_hbm.at[pl.ds(base, CH)], eav[b])
            cps.append((pltpu.async_copy(p_hbm.at[srcv[b]], pbuf[b], semp[b]),
                        pltpu.async_copy(dstt_hbm.at[dstv[b]], dbuf[b], semd[b])))
        scs = []
        for b in range(NB):
            cps[b][0].wait()
            cps[b][1].wait()

            def group(g, _, b=b):
                rows = iota + g * 16
                eaj = [plsc.load_gather(
                    eav[b], [rows, jnp.full((16,), j, jnp.int32)])
                    for j in range(EDGE_D)]
                for hh in range(2):
                    hcol = hbase + hh
                    als = plsc.load_gather(
                        pbuf[b], [rows, jnp.full((16,), HC, jnp.int32) + hcol])
                    ald = plsc.load_gather(
                        dbuf[b], [rows, jnp.full((16,), 0, jnp.int32) + hcol])
                    kv = plsc.load_gather(
                        dbuf[b], [rows, jnp.full((16,), H, jnp.int32) + hcol])
                    ale = eaj[0] * wsp[hh][0]
                    for j in range(1, EDGE_D):
                        ale = ale + eaj[j] * wsp[hh][j]
                    z = als + ald + ale
                    alpha = jnp.where(z > 0, z, 0.2 * z)
                    ex = jnp.exp(alpha - kv)
                    plsc.store_scatter(
                        msg[b], [rows, jnp.full((16,), 2 * C + hh, jnp.int32)],
                        ex)
                    for c in range(C):
                        hv = plsc.load_gather(
                            pbuf[b],
                            [rows, jnp.full((16,), c, jnp.int32) + hcol * C])
                        plsc.store_scatter(
                            msg[b],
                            [rows, jnp.full((16,), hh * C + c, jnp.int32)],
                            ex * hv)
                return 0

            lax.fori_loop(0, CH // 16, group, 0)
            pltpu.sync_copy(msg[b], acc.at[dstv[b]], add=True)
        return 0

    lax.fori_loop(0, CPT // NB, superchunk, 0)
    plsc.subcore_barrier()
    pltpu.sync_copy(acc.at[pl.ds(sid * ZSL, ZSL)],
                    out_hbm.at[cid, pl.ds(sid * ZSL, ZSL)])


@functools.lru_cache(maxsize=1)
def _sc_kernels():
    mesh = plsc.VectorSubcoreMesh(core_axis_name="c", subcore_axis_name="s")
    cp = pltpu.CompilerParams(needs_layout_passes=False,
                              use_tc_tiling_on_sc=False)
    pass_b = pl.kernel(
        _pass_b_body,
        out_type=jax.ShapeDtypeStruct((2, NPAD, 24), jnp.float32),
        mesh=mesh,
        compiler_params=cp,
        scratch_types=[
            [pltpu.VMEM((CH,), jnp.int32)] * NB,
            [pltpu.VMEM((CH,), jnp.int32)] * NB,
            [pltpu.VMEM((CH, EDGE_D), jnp.float32)] * NB,
            [pltpu.VMEM((CH, 40), jnp.float32)] * NB,
            [pltpu.VMEM((CH, 8), jnp.float32)] * NB,
            [pltpu.VMEM((CH, 24), jnp.float32)] * NB,
            pltpu.VMEM((32,), jnp.float32),
            pltpu.VMEM_SHARED((NPAD, 24), jnp.float32),
            [pltpu.SemaphoreType.DMA] * NB,
            [pltpu.SemaphoreType.DMA] * NB,
            [pltpu.SemaphoreType.DMA] * NB,
        ],
    )
    return pass_b


def _fold(a):
    idx = jnp.arange(HC)
    return jnp.zeros((HC, H), jnp.float32).at[idx, idx // C].set(a.reshape(HC))


def kernel(x, edge_index, edge_attr, batch,
           W1, as1, ad1, We1, ae1, b1,
           W2, as2, ad2, We2, ae2, b2,
           W3, as3, ad3, We3, ae3, b3,
           Wm1, bm1, Wm2, bm2, Wm3, bm3):
    src = edge_index[0].astype(jnp.int32)
    dst = edge_index[1].astype(jnp.int32)
    src_p = jnp.concatenate([src, jnp.zeros((EPAD - E,), jnp.int32)])
    dst_p = jnp.concatenate([dst, jnp.full((EPAD - E,), N, jnp.int32)])
    ea_p = jnp.concatenate(
        [edge_attr, jnp.zeros((EPAD - E, EDGE_D), jnp.float32)])

    st = _ea_stats(edge_attr)
    m_abs = st[0, :EDGE_D]
    fill = st[0, EDGE_D:] / E

    z24 = jnp.zeros((NPAD, 24), jnp.float32)

    pass_b = _sc_kernels()
    layers = [(W1, as1, ad1, We1, ae1, b1),
              (W2, as2, ad2, We2, ae2, b2),
              (W3, as3, ad3, We3, ae3, b3)]
    x_cur = x[:, :NODE_D]
    f3 = None
    for li, (W, a_s, a_d, We, a_e, b) in enumerate(layers):
        wfull = jnp.concatenate(
            [W, W @ _fold(a_s), W @ _fold(a_d)], axis=1)
        we_a = jnp.einsum('jhc,hc->jh', We.reshape(EDGE_D, H, C), a_e)
        p_tab, s8, bm = _prep(x_cur, wfull)
        alsmax = jnp.max(bm[:, 0, :H], axis=0)
        ub = jnp.sum(m_abs[:, None] * jnp.abs(we_a), axis=0)
        bub = alsmax + ub
        al_fill = fill @ we_a
        weab = jnp.concatenate(
            [we_a.reshape(-1), bub, jnp.zeros((12,), jnp.float32)])
        aux = jnp.concatenate(
            [bub, al_fill, jnp.zeros((8,), jnp.float32)]).reshape(1, 16)
        dstt, exl = _k_kernel(s8, aux)
        accp = pass_b(src_p, dst_p, ea_p, p_tab, dstt, weab, z24)
        if li < 2:
            x_cur = _combine12(accp, exl, p_tab, b.reshape(1, HC))
        else:
            f3 = _combine3(accp, exl, p_tab, b.reshape(1, C))

    gf = x[:, NODE_D:]
    ids3 = batch.astype(jnp.int32).reshape(GRID_N, 1, BN)
    return _pool_mlp(f3, gf, ids3,
                     Wm1, bm1.reshape(1, C),
                     Wm2, bm2.reshape(1, C // 2),
                     Wm3, bm3.reshape(1, OUT))


# speedup vs baseline: 1.5201x; 1.0257x over previous
import functools

import jax
import jax.numpy as jnp
from jax import lax
from jax.experimental import pallas as pl
from jax.experimental.pallas import tpu as pltpu
from jax.experimental.pallas import tpu_sc as plsc

N = 50000
E = 800000
F_IN = 128
NODE_D = 112
GRAPH_D = 16
H = 4
C = 8
EDGE_D = 4
OUT = 8
G = 64
HC = H * C

NPAD = 50048
ZSL = NPAD // 16
CH = 256
CPT = 196
EPAD = 16 * CPT * CH
BN = 1000
GRID_N = N // BN


def _ea_stats_body(ea_ref, st_ref):
    i = pl.program_id(0)
    blk = ea_ref[...]
    m = jnp.max(jnp.abs(blk), axis=0, keepdims=True)
    s = jnp.sum(blk, axis=0, keepdims=True)

    @pl.when(i == 0)
    def _():
        st_ref[...] = jnp.concatenate([m, s], axis=1)

    @pl.when(i > 0)
    def _():
        st_ref[...] = jnp.concatenate(
            [jnp.maximum(st_ref[:, :EDGE_D], m), st_ref[:, EDGE_D:] + s], axis=1)


def _ea_stats(ea):
    eb = 8000
    return pl.pallas_call(
        _ea_stats_body,
        grid=(E // eb,),
        in_specs=[pl.BlockSpec((eb, EDGE_D), lambda i: (i, 0))],
        out_specs=pl.BlockSpec((1, 2 * EDGE_D), lambda i: (0, 0)),
        out_shape=jax.ShapeDtypeStruct((1, 2 * EDGE_D), jnp.float32),
    )(ea)


def _prep_body(x_ref, w_ref, p_ref, s8_ref, bm_ref):
    r = jnp.dot(x_ref[...], w_ref[...], preferred_element_type=jnp.float32)
    p_ref[...] = r
    s8_ref[...] = r[:, HC:HC + 8]
    bm_ref[...] = jnp.max(r[:, HC:HC + 8], axis=0, keepdims=True)[None]


def _prep(x_cur, wfull):
    din = x_cur.shape[1]
    return pl.pallas_call(
        _prep_body,
        grid=(GRID_N,),
        in_specs=[pl.BlockSpec((BN, din), lambda i: (i, 0)),
                  pl.BlockSpec((din, 40), lambda i: (0, 0))],
        out_specs=[pl.BlockSpec((BN, 40), lambda i: (i, 0)),
                   pl.BlockSpec((BN, 8), lambda i: (i, 0)),
                   pl.BlockSpec((1, 1, 8), lambda i: (i, 0, 0))],
        out_shape=[jax.ShapeDtypeStruct((N, 40), jnp.float32),
                   jax.ShapeDtypeStruct((N, 8), jnp.float32),
                   jax.ShapeDtypeStruct((GRID_N, 1, 8), jnp.float32)],
    )(x_cur, wfull)


def _k_body(s8_ref, aux_ref, dst_ref, exl_ref):
    s8 = s8_ref[...]
    als = s8[:, :H]
    ald = s8[:, H:2 * H]
    bub = aux_ref[0, :H]
    alf = aux_ref[0, H:2 * H]
    beta_self = als + alf[None, :]
    kz = bub[None, :] + ald
    kv = jnp.where(kz > 0, kz, 0.2 * kz)
    zl = beta_self + ald
    al_loop = jnp.where(zl > 0, zl, 0.2 * zl)
    exl = jnp.exp(al_loop - kv)
    zpad = jnp.zeros_like(als)
    dst_ref[...] = jnp.concatenate([ald, kv], axis=1)
    exl_ref[...] = jnp.concatenate([exl, zpad], axis=1)


def _k_kernel(s8, aux):
    kb = 1024
    grid = pl.cdiv(NPAD, kb)
    return pl.pallas_call(
        _k_body,
        grid=(grid,),
        in_specs=[pl.BlockSpec((kb, 8), lambda i: (i, 0)),
                  pl.BlockSpec((1, 16), lambda i: (0, 0))],
        out_specs=[pl.BlockSpec((kb, 8), lambda i: (i, 0)),
                   pl.BlockSpec((kb, 8), lambda i: (i, 0))],
        out_shape=[jax.ShapeDtypeStruct((NPAD, 8), jnp.float32),
                   jax.ShapeDtypeStruct((NPAD, 8), jnp.float32)],
    )(s8, aux)


def _combine12_body(a_ref, e_ref, p_ref, b_ref, x_ref):
    hmat = p_ref[:, :HC]
    exl = e_ref[...]
    outs = []
    for h in range(H):
        core, loc = divmod(h, 2)
        num = a_ref[core][:, loc * C:(loc + 1) * C]
        den = a_ref[core][:, 2 * C + loc:2 * C + loc + 1]
        el = exl[:, h:h + 1]
        hh = hmat[:, h * C:(h + 1) * C]
        outs.append((num + el * hh) / (den + el))
    o = jnp.concatenate(outs, axis=1) + b_ref[0][None, :]
    x_ref[...] = jnp.maximum(o, 0.0)


def _combine12(accp, exl, p, b):
    return pl.pallas_call(
        _combine12_body,
        grid=(GRID_N,),
        in_specs=[pl.BlockSpec((2, BN, 24), lambda i: (0, i, 0)),
                  pl.BlockSpec((BN, 8), lambda i: (i, 0)),
                  pl.BlockSpec((BN, 40), lambda i: (i, 0)),
                  pl.BlockSpec((1, HC), lambda i: (0, 0))],
        out_specs=pl.BlockSpec((BN, HC), lambda i: (i, 0)),
        out_shape=jax.ShapeDtypeStruct((N, HC), jnp.float32),
    )(accp, exl, p, b)


def _combine3_body(a_ref, e_ref, p_ref, b_ref, f_ref):
    hmat = p_ref[:, :HC]
    exl = e_ref[...]
    acc = None
    for h in range(H):
        core, loc = divmod(h, 2)
        num = a_ref[core][:, loc * C:(loc + 1) * C]
        den = a_ref[core][:, 2 * C + loc:2 * C + loc + 1]
        el = exl[:, h:h + 1]
        hh = hmat[:, h * C:(h + 1) * C]
        o = (num + el * hh) / (den + el)
        acc = o if acc is None else acc + o
    f_ref[...] = acc * (1.0 / H) + b_ref[0][None, :]


def _combine3(accp, exl, p, b3):
    return pl.pallas_call(
        _combine3_body,
        grid=(GRID_N,),
        in_specs=[pl.BlockSpec((2, BN, 24), lambda i: (0, i, 0)),
                  pl.BlockSpec((BN, 8), lambda i: (i, 0)),
                  pl.BlockSpec((BN, 40), lambda i: (i, 0)),
                  pl.BlockSpec((1, C), lambda i: (0, 0))],
        out_specs=pl.BlockSpec((BN, C), lambda i: (i, 0)),
        out_shape=jax.ShapeDtypeStruct((N, C), jnp.float32),
    )(accp, exl, p, b3)


def _pool_body(f_ref, gf_ref, ids_ref, wm1_ref, bm1_ref, wm2_ref, bm2_ref,
               wm3_ref, bm3_ref, o_ref, ps_ref):
    i = pl.program_id(0)
    ids = ids_ref[0]
    giota = lax.broadcasted_iota(jnp.int32, (G, BN), 0)
    oh = (giota == ids).astype(jnp.float32)
    feat = jnp.concatenate(
        [f_ref[...], gf_ref[...], jnp.ones((BN, 1), jnp.float32)], axis=1)
    part = jnp.dot(oh, feat, preferred_element_type=jnp.float32)

    @pl.when(i == 0)
    def _():
        ps_ref[...] = part

    @pl.when(i > 0)
    def _():
        ps_ref[...] = ps_ref[...] + part

    @pl.when(i == GRID_N - 1)
    def _():
        ps = ps_ref[...]
        cnt = jnp.maximum(ps[:, 24:25], 1.0)
        z = ps[:, :24] / cnt
        z = jnp.maximum(jnp.dot(z, wm1_ref[...], preferred_element_type=jnp.float32)
                        + bm1_ref[0][None, :], 0.0)
        z = jnp.maximum(jnp.dot(z, wm2_ref[...], preferred_element_type=jnp.float32)
                        + bm2_ref[0][None, :], 0.0)
        o_ref[...] = (jnp.dot(z, wm3_ref[...], preferred_element_type=jnp.float32)
                      + bm3_ref[0][None, :])


def _pool_mlp(f, gf, ids3, Wm1, bm1, Wm2, bm2, Wm3, bm3):
    return pl.pallas_call(
        _pool_body,
        grid=(GRID_N,),
        in_specs=[pl.BlockSpec((BN, C), lambda i: (i, 0)),
                  pl.BlockSpec((BN, GRAPH_D), lambda i: (i, 0)),
                  pl.BlockSpec((1, 1, BN), lambda i: (i, 0, 0)),
                  pl.BlockSpec((C + GRAPH_D, C), lambda i: (0, 0)),
                  pl.BlockSpec((1, C), lambda i: (0, 0)),
                  pl.BlockSpec((C, C // 2), lambda i: (0, 0)),
                  pl.BlockSpec((1, C // 2), lambda i: (0, 0)),
                  pl.BlockSpec((C // 2, OUT), lambda i: (0, 0)),
                  pl.BlockSpec((1, OUT), lambda i: (0, 0))],
        out_specs=pl.BlockSpec((G, OUT), lambda i: (0, 0)),
        out_shape=jax.ShapeDtypeStruct((G, OUT), jnp.float32),
        scratch_shapes=[pltpu.VMEM((G, 25), jnp.float32)],
    )(f, gf, ids3, Wm1, bm1, Wm2, bm2, Wm3, bm3)


def _splat_dyn(ref, base, off):
    return plsc.load_gather(ref, [jnp.full((16,), base, jnp.int32) + off])


NB = 2


def _pass_b_body(src_hbm, dst_hbm, ea_hbm, p_hbm, dstt_hbm, weab_hbm, z_hbm,
                 out_hbm, srcv, dstv, eav, pbuf, dbuf, msg, weav, acc,
                 semp, semd, sems):
    cid = lax.axis_index("c")
    sid = lax.axis_index("s")
    pltpu.sync_copy(z_hbm.at[pl.ds(sid * ZSL, ZSL)], acc.at[pl.ds(sid * ZSL, ZSL)])
    pltpu.sync_copy(weab_hbm, weav)

    def zrow(i, _):
        for b in range(NB):
            msg[b][i, pl.ds(0, 16)] = jnp.zeros((16,), jnp.float32)
            msg[b][i, pl.ds(8, 16)] = jnp.zeros((16,), jnp.float32)
        return 0

    lax.fori_loop(0, CH, zrow, 0)
    plsc.subcore_barrier()

    iota = lax.iota(jnp.int32, 16)
    hbase = cid * 2
    wsp = [[_splat_dyn(weav, j * H, hbase + hh) for j in range(EDGE_D)]
           for hh in range(2)]

    def superchunk(i, _):
        cps = []
        for b in range(NB):
            base = (sid * CPT + i * NB + b) * CH
            pltpu.sync_copy(src_hbm.at[pl.ds(base, CH)], srcv[b])
            pltpu.sync_copy(dst_hbm.at[pl.ds(base, CH)], dstv[b])
            pltpu.sync_copy(ea_hbm.at[pl.ds(base, CH)], eav[b])
            cps.append((pltpu.async_copy(p_hbm.at[srcv[b]], pbuf[b], semp[b]),
                        pltpu.async_copy(dstt_hbm.at[dstv[b]], dbuf[b], semd[b])))
        scs = []
        for b in range(NB):
            cps[b][0].wait()
            cps[b][1].wait()

            def group(g, _, b=b):
                rows = iota + g * 16
                eaj = [plsc.load_gather(
                    eav[b], [rows, jnp.full((16,), j, jnp.int32)])
                    for j in range(EDGE_D)]
                for hh in range(2):
                    hcol = hbase + hh
                    als = plsc.load_gather(
                        pbuf[b], [rows, jnp.full((16,), HC, jnp.int32) + hcol])
                    ald = plsc.load_gather(
                        dbuf[b], [rows, jnp.full((16,), 0, jnp.int32) + hcol])
                    kv = plsc.load_gather(
                        dbuf[b], [rows, jnp.full((16,), H, jnp.int32) + hcol])
                    ale = eaj[0] * wsp[hh][0]
                    for j in range(1, EDGE_D):
                        ale = ale + eaj[j] * wsp[hh][j]
                    z = als + ald + ale
                    alpha = jnp.where(z > 0, z, 0.2 * z)
                    ex = jnp.exp(alpha - kv)
                    plsc.store_scatter(
                        msg[b], [rows, jnp.full((16,), 2 * C + hh, jnp.int32)],
                        ex)
                    for c in range(C):
                        hv = plsc.load_gather(
                            pbuf[b],
                            [rows, jnp.full((16,), c, jnp.int32) + hcol * C])
                        plsc.store_scatter(
                            msg[b],
                            [rows, jnp.full((16,), hh * C + c, jnp.int32)],
                            ex * hv)
                return 0

            lax.fori_loop(0, CH // 16, group, 0)
            pltpu.sync_copy(msg[b], acc.at[dstv[b]], add=True)
        return 0

    lax.fori_loop(0, CPT // NB, superchunk, 0)
    plsc.subcore_barrier()
    pltpu.sync_copy(acc.at[pl.ds(sid * ZSL, ZSL)],
                    out_hbm.at[cid, pl.ds(sid * ZSL, ZSL)])


@functools.lru_cache(maxsize=1)
def _sc_kernels():
    mesh = plsc.VectorSubcoreMesh(core_axis_name="c", subcore_axis_name="s")
    cp = pltpu.CompilerParams(needs_layout_passes=False,
                              use_tc_tiling_on_sc=False)
    pass_b = pl.kernel(
        _pass_b_body,
        out_type=jax.ShapeDtypeStruct((2, NPAD, 24), jnp.float32),
        mesh=mesh,
        compiler_params=cp,
        scratch_types=[
            [pltpu.VMEM((CH,), jnp.int32)] * NB,
            [pltpu.VMEM((CH,), jnp.int32)] * NB,
            [pltpu.VMEM((CH, EDGE_D), jnp.float32)] * NB,
            [pltpu.VMEM((CH, 40), jnp.float32)] * NB,
            [pltpu.VMEM((CH, 8), jnp.float32)] * NB,
            [pltpu.VMEM((CH, 24), jnp.float32)] * NB,
            pltpu.VMEM((32,), jnp.float32),
            pltpu.VMEM_SHARED((NPAD, 24), jnp.float32),
            [pltpu.SemaphoreType.DMA] * NB,
            [pltpu.SemaphoreType.DMA] * NB,
            [pltpu.SemaphoreType.DMA] * NB,
        ],
    )
    return pass_b


def _fold(a):
    idx = jnp.arange(HC)
    return jnp.zeros((HC, H), jnp.float32).at[idx, idx // C].set(a.reshape(HC))


def kernel(x, edge_index, edge_attr, batch,
           W1, as1, ad1, We1, ae1, b1,
           W2, as2, ad2, We2, ae2, b2,
           W3, as3, ad3, We3, ae3, b3,
           Wm1, bm1, Wm2, bm2, Wm3, bm3):
    src = edge_index[0].astype(jnp.int32)
    dst = edge_index[1].astype(jnp.int32)
    src_p = jnp.concatenate([src, jnp.zeros((EPAD - E,), jnp.int32)])
    dst_p = jnp.concatenate([dst, jnp.full((EPAD - E,), N, jnp.int32)])
    ea_p = jnp.concatenate(
        [edge_attr, jnp.zeros((EPAD - E, EDGE_D), jnp.float32)])

    st = _ea_stats(edge_attr)
    m_abs = st[0, :EDGE_D]
    fill = st[0, EDGE_D:] / E

    z24 = jnp.zeros((NPAD, 24), jnp.float32)

    pass_b = _sc_kernels()
    layers = [(W1, as1, ad1, We1, ae1, b1),
              (W2, as2, ad2, We2, ae2, b2),
              (W3, as3, ad3, We3, ae3, b3)]
    x_cur = x[:, :NODE_D]
    f3 = None
    for li, (W, a_s, a_d, We, a_e, b) in enumerate(layers):
        wfull = jnp.concatenate(
            [W, W @ _fold(a_s), W @ _fold(a_d)], axis=1)
        we_a = jnp.einsum('jhc,hc->jh', We.reshape(EDGE_D, H, C), a_e)
        p_tab, s8, bm = _prep(x_cur, wfull)
        alsmax = jnp.max(bm[:, 0, :H], axis=0)
        ub = jnp.sum(m_abs[:, None] * jnp.abs(we_a), axis=0)
        bub = alsmax + ub
        al_fill = fill @ we_a
        weab = jnp.concatenate(
            [we_a.reshape(-1), bub, jnp.zeros((12,), jnp.float32)])
        aux = jnp.concatenate(
            [bub, al_fill, jnp.zeros((8,), jnp.float32)]).reshape(1, 16)
        dstt, exl = _k_kernel(s8, aux)
        accp = pass_b(src_p, dst_p, ea_p, p_tab, dstt, weab, z24)
        if li < 2:
            x_cur = _combine12(accp, exl, p_tab, b.reshape(1, HC))
        else:
            f3 = _combine3(accp, exl, p_tab, b.reshape(1, C))

    gf = x[:, NODE_D:]
    ids3 = batch.astype(jnp.int32).reshape(GRID_N, 1, BN)
    return _pool_mlp(f3, gf, ids3,
                     Wm1, bm1.reshape(1, C),
                     Wm2, bm2.reshape(1, C // 2),
                     Wm3, bm3.reshape(1, OUT))
